# baseline (speedup 1.0000x reference)
import functools

import jax
import jax.numpy as jnp
from jax.experimental import pallas as pl
from jax.experimental.pallas import tpu as pltpu
from jax.experimental.pallas import tpu_sc as plsc

F32 = jnp.float32
INF = float('inf')


def _params(grid_len):
    return pltpu.CompilerParams(
        dimension_semantics=("parallel",) * grid_len)


def _fps_body(xyz_ref, out_ref, *, npoint, n, b):
    x = xyz_ref[:, 0, :]
    y = xyz_ref[:, 1, :]
    z = xyz_ref[:, 2, :]
    iota = jax.lax.broadcasted_iota(jnp.int32, (b, n), 1)

    def body(i, carry):
        dists, far = carry
        mask = iota == far
        cx = jnp.sum(jnp.where(mask, x, 0.0), axis=1)
        cy = jnp.sum(jnp.where(mask, y, 0.0), axis=1)
        cz = jnp.sum(jnp.where(mask, z, 0.0), axis=1)
        c = jnp.concatenate([cx[:, None], cy[:, None], cz[:, None]], axis=1)
        out_ref[0, pl.ds(i, 1)] = c[None]
        d = (x - cx[:, None]) ** 2
        d = d + (y - cy[:, None]) ** 2
        d = d + (z - cz[:, None]) ** 2
        dists = jnp.minimum(dists, d)
        far = jnp.argmax(dists, axis=1).astype(jnp.int32)[:, None]
        return dists, far

    dists0 = jnp.full((b, n), 1e10, F32)
    far0 = jnp.zeros((b, 1), jnp.int32)
    jax.lax.fori_loop(0, npoint, body, (dists0, far0))


def _fps(xyz, npoint):
    b, _, n = xyz.shape
    out = pl.pallas_call(
        functools.partial(_fps_body, npoint=npoint, n=n, b=b),
        grid=(1,),
        in_specs=[pl.BlockSpec((b, 3, n), lambda i: (0, 0, 0))],
        out_specs=pl.BlockSpec((1, npoint, b, 3), lambda i: (0, 0, 0, 0)),
        out_shape=jax.ShapeDtypeStruct((1, npoint, b, 3), F32),
    )(xyz)
    return jnp.transpose(out, (0, 2, 3, 1)).reshape(b, 3, npoint)


def _knn_body(q_ref, p_ref, idx_ref, *, k, n, t):
    q = q_ref[0]
    p = p_ref[0]
    q2 = jnp.sum(q * q, axis=1)[:, None]
    p2 = jnp.sum(p * p, axis=1)[None, :]
    d = -2.0 * jnp.dot(q, p.T, preferred_element_type=F32)
    d = d + q2
    d = d + p2
    iota = jax.lax.broadcasted_iota(jnp.int32, (t, n), 1)
    cols = []
    for _ in range(k):
        j = jnp.argmin(d, axis=1).astype(jnp.int32)[:, None]
        cols.append(j)
        d = jnp.where(iota == j, INF, d)
    idx_ref[0] = jnp.concatenate(cols, axis=1) + pl.program_id(0) * n


def _knn(new_xyz, xyz, k=16, t=1024):
    b, _, s = new_xyz.shape
    n = xyz.shape[2]
    t = min(t, s)
    qt = jnp.transpose(new_xyz, (0, 2, 1))
    pt = jnp.transpose(xyz, (0, 2, 1))
    return pl.pallas_call(
        functools.partial(_knn_body, k=k, n=n, t=t),
        grid=(b, s // t),
        in_specs=[
            pl.BlockSpec((1, t, 3), lambda i, j: (i, j, 0)),
            pl.BlockSpec((1, n, 3), lambda i, j: (i, 0, 0)),
        ],
        out_specs=pl.BlockSpec((1, t, k), lambda i, j: (i, j, 0)),
        out_shape=jax.ShapeDtypeStruct((b, s, k), jnp.int32),
        compiler_params=_params(2),
    )(qt, pt)


def _sc_gather(table, idx):
    v, d = table.shape
    bsz = idx.shape[0]
    info = plsc.get_sparse_core_info()
    nc = info.num_cores
    nw = nc * info.num_subcores
    b_per_w = bsz // nw
    c = 256 if d <= 128 else 128
    iters = b_per_w // (2 * c)
    mesh = plsc.VectorSubcoreMesh(core_axis_name="c", subcore_axis_name="s")

    @functools.partial(
        pl.kernel, mesh=mesh,
        out_type=jax.ShapeDtypeStruct((bsz, d), F32),
        scratch_types=[
            pltpu.VMEM((c,), jnp.int32),
            pltpu.VMEM((c, d), F32),
            pltpu.VMEM((c,), jnp.int32),
            pltpu.VMEM((c, d), F32),
            pltpu.SemaphoreType.DMA,
            pltpu.SemaphoreType.DMA,
        ],
    )
    def k(table_hbm, idx_hbm, out_hbm, idx_v0, rows_v0, idx_v1, rows_v1,
          sem0, sem1):
        wid = jax.lax.axis_index("s") * nc + jax.lax.axis_index("c")
        base = wid * b_per_w

        def body(it, carry):
            off0 = base + (2 * it) * c
            off1 = off0 + c
            pltpu.sync_copy(idx_hbm.at[pl.ds(off0, c)], idx_v0)
            h0 = pltpu.async_copy(table_hbm.at[idx_v0], rows_v0, sem0)
            pltpu.sync_copy(idx_hbm.at[pl.ds(off1, c)], idx_v1)
            h1 = pltpu.async_copy(table_hbm.at[idx_v1], rows_v1, sem1)
            h0.wait()
            pltpu.sync_copy(rows_v0, out_hbm.at[pl.ds(off0, c)])
            h1.wait()
            pltpu.sync_copy(rows_v1, out_hbm.at[pl.ds(off1, c)])
            return carry

        jax.lax.fori_loop(0, iters, body, 0)

    return k(table, idx)


def _proj_body(table_ref, w1_ref, out_ref, *, pad):
    h = jnp.dot(table_ref[0], w1_ref[:], preferred_element_type=F32)
    if pad:
        h = jnp.concatenate([h, jnp.zeros((h.shape[0], pad), F32)], axis=1)
    out_ref[0] = h


def _proj(table, w1):
    b, n, f = table.shape
    h1 = w1.shape[1]
    h1p = -(-h1 // 128) * 128
    return pl.pallas_call(
        functools.partial(_proj_body, pad=h1p - h1),
        grid=(b,),
        in_specs=[
            pl.BlockSpec((1, n, f), lambda i: (i, 0, 0)),
            pl.BlockSpec((f, h1), lambda i: (0, 0)),
        ],
        out_specs=pl.BlockSpec((1, n, h1p), lambda i: (i, 0, 0)),
        out_shape=jax.ShapeDtypeStruct((b, n, h1p), F32),
        compiler_params=_params(1),
    )(table, w1)


def _sa_post_body(gath_ref, center_ref, w1x_ref, b1_ref, w2_ref, b2_ref,
                  out_ref, *, s, k, t):
    w1x = w1x_ref[:]
    b1 = b1_ref[:]
    w2 = w2_ref[:]
    b2 = b2_ref[:]
    h1 = w1x.shape[1]
    for ti in range(s // t):
        lo = ti * t
        g = gath_ref[0, lo * k:(lo + t) * k, 0:h1]
        center_t = center_ref[0, lo:lo + t, :]
        c1 = jnp.dot(center_t, w1x, preferred_element_type=F32)
        h = jnp.maximum(
            (g.reshape(t, k, h1) - c1[:, None, :]).reshape(t * k, h1)
            + b1, 0.0)
        h2 = jnp.dot(h, w2, preferred_element_type=F32) + b2
        out_ref[0, lo:lo + t, :] = jnp.max(
            h2.reshape(t, k, h2.shape[1]), axis=1)


def _sa(table, center, idx, p, t):
    b, n, f = table.shape
    s = center.shape[1]
    k = idx.shape[2]
    w1 = p['w1'].T
    w2 = p['w2'].T
    h1 = w1.shape[1]
    h2 = w2.shape[1]
    proj = _proj(table, w1)
    h1p = proj.shape[2]
    gath = _sc_gather(proj.reshape(b * n, h1p), idx.reshape(-1))
    gath = gath.reshape(b, s * k, h1p)
    return pl.pallas_call(
        functools.partial(_sa_post_body, s=s, k=k, t=t),
        grid=(b,),
        in_specs=[
            pl.BlockSpec((1, s * k, h1p), lambda i: (i, 0, 0)),
            pl.BlockSpec((1, s, 3), lambda i: (i, 0, 0)),
            pl.BlockSpec((3, h1), lambda i: (0, 0)),
            pl.BlockSpec((1, h1), lambda i: (0, 0)),
            pl.BlockSpec((h1, h2), lambda i: (0, 0)),
            pl.BlockSpec((1, h2), lambda i: (0, 0)),
        ],
        out_specs=pl.BlockSpec((1, s, h2), lambda i: (i, 0, 0)),
        out_shape=jax.ShapeDtypeStruct((b, s, h2), F32),
        compiler_params=_params(1),
    )(gath, center, jnp.asarray(w1)[0:3, :], p['b1'].reshape(1, -1), w2,
      p['b2'].reshape(1, -1))


def _vt_pre_body(x_ref, pos_ref, ls_w_ref, ls_b_ref, k_w_ref, k_b_ref,
                 q_w_ref, q_b_ref, v_w_ref, v_b_ref, pw1_ref,
                 tab_ref, qp_ref):
    x = x_ref[0]
    xl = jnp.dot(x, ls_w_ref[:], preferred_element_type=F32) + ls_b_ref[:]
    key = jnp.dot(xl, k_w_ref[:], preferred_element_type=F32) + k_b_ref[:]
    val = jnp.dot(xl, v_w_ref[:], preferred_element_type=F32) + v_b_ref[:]
    qry = jnp.dot(xl, q_w_ref[:], preferred_element_type=F32) + q_b_ref[:]
    pp = jnp.dot(pos_ref[0], pw1_ref[:], preferred_element_type=F32)
    pad = jnp.zeros((pp.shape[0], 64), F32)
    tab_ref[0] = jnp.concatenate([key, val, pp, pad], axis=1)
    qp_ref[0] = jnp.concatenate([qry, pp], axis=1)


def _vt_post_body(x_ref, qp_ref, gath_ref,
                  pb1_ref, pg_ref, pbb_ref, pw2_ref, pb2_ref,
                  aw1_ref, ab1_ref, ag_ref, abb_ref, aw2_ref, ab2_ref,
                  le_w_ref, le_b_ref, out_ref, agg_s, *, np_, c, k, t):
    for ti in range(np_ // t):
        lo = ti * t
        g = gath_ref[0, lo * k:(lo + t) * k, :]
        kg = g[:, 0:64]
        vg = g[:, 64:128]
        ppg = g[:, 128:192]
        qp_t = qp_ref[0, lo:lo + t, :]
        q_t = qp_t[:, 0:64]
        pp_t = qp_t[:, 64:128]
        qk_rel = (q_t[:, None, :] - kg.reshape(t, k, 64)).reshape(t * k, 64)
        pe = (pp_t[:, None, :] - ppg.reshape(t, k, 64)).reshape(t * k, 64) \
            + pb1_ref[:]
        pe = jnp.maximum(pe * pg_ref[:] + pbb_ref[:], 0.0)
        pe = jnp.dot(pe, pw2_ref[:], preferred_element_type=F32) + pb2_ref[:]
        a = jnp.dot(qk_rel + pe, aw1_ref[:], preferred_element_type=F32) \
            + ab1_ref[:]
        a = jnp.maximum(a * ag_ref[:] + abb_ref[:], 0.0)
        a = jnp.dot(a, aw2_ref[:], preferred_element_type=F32) + ab2_ref[:]
        a3 = a.reshape(t, k, 64)
        mx = jnp.max(a3, axis=1, keepdims=True)
        e = jnp.exp(a3 - mx)
        sm = e / jnp.sum(e, axis=1, keepdims=True)
        vpe = (vg + pe).reshape(t, k, 64)
        agg_s[lo:lo + t, :] = jnp.sum(sm * vpe, axis=1)
    out_ref[0] = jnp.dot(agg_s[:], le_w_ref[:],
                         preferred_element_type=F32) + le_b_ref[:] + x_ref[0]


def _vt(x, pos, idx, p, t=256):
    b, np_, c = x.shape
    k = idx.shape[2]
    pre_ws = [p['ls_w'].T, p['ls_b'].reshape(1, -1),
              p['k_w'].T, p['k_b'].reshape(1, -1),
              p['q_w'].T, p['q_b'].reshape(1, -1),
              p['v_w'].T, p['v_b'].reshape(1, -1),
              p['pos_w1'].T]
    pre_specs = [pl.BlockSpec(w.shape, lambda i: (0, 0)) for w in pre_ws]
    tab, qp = pl.pallas_call(
        _vt_pre_body,
        grid=(b,),
        in_specs=[
            pl.BlockSpec((1, np_, c), lambda i: (i, 0, 0)),
            pl.BlockSpec((1, np_, 3), lambda i: (i, 0, 0)),
        ] + pre_specs,
        out_specs=[
            pl.BlockSpec((1, np_, 256), lambda i: (i, 0, 0)),
            pl.BlockSpec((1, np_, 128), lambda i: (i, 0, 0)),
        ],
        out_shape=[
            jax.ShapeDtypeStruct((b, np_, 256), F32),
            jax.ShapeDtypeStruct((b, np_, 128), F32),
        ],
        compiler_params=_params(1),
    )(x, pos, *pre_ws)

    gath = _sc_gather(tab.reshape(b * np_, 256), idx.reshape(-1))
    gath = gath.reshape(b, np_ * k, 256)

    post_ws = [p['pos_b1'].reshape(1, -1),
               p['pos_bn_g'].reshape(1, -1), p['pos_bn_b'].reshape(1, -1),
               p['pos_w2'].T, p['pos_b2'].reshape(1, -1),
               p['attn_w1'].T, p['attn_b1'].reshape(1, -1),
               p['attn_bn_g'].reshape(1, -1), p['attn_bn_b'].reshape(1, -1),
               p['attn_w2'].T, p['attn_b2'].reshape(1, -1),
               p['le_w'].T, p['le_b'].reshape(1, -1)]
    post_specs = [pl.BlockSpec(w.shape, lambda i: (0, 0)) for w in post_ws]
    return pl.pallas_call(
        functools.partial(_vt_post_body, np_=np_, c=c, k=k, t=t),
        grid=(b,),
        in_specs=[
            pl.BlockSpec((1, np_, c), lambda i: (i, 0, 0)),
            pl.BlockSpec((1, np_, 128), lambda i: (i, 0, 0)),
            pl.BlockSpec((1, np_ * k, 256), lambda i: (i, 0, 0)),
        ] + post_specs,
        out_specs=pl.BlockSpec((1, np_, c), lambda i: (i, 0, 0)),
        out_shape=jax.ShapeDtypeStruct((b, np_, c), F32),
        scratch_shapes=[pltpu.VMEM((np_, 64), F32)],
        compiler_params=_params(1),
    )(x, qp, gath, *post_ws)


def _sa3_body(table_ref, w1_ref, b1_ref, w2_ref, b2_ref, out_ref):
    h = jnp.maximum(
        jnp.dot(table_ref[0], w1_ref[:], preferred_element_type=F32)
        + b1_ref[:], 0.0)
    h2 = jnp.dot(h, w2_ref[:], preferred_element_type=F32) + b2_ref[:]
    out_ref[0] = jnp.max(h2, axis=0, keepdims=True)


def _sa3(table, p):
    b, n, f = table.shape
    w1 = p['w1'].T
    w2 = p['w2'].T
    h1 = w1.shape[1]
    h2 = w2.shape[1]
    out = pl.pallas_call(
        _sa3_body,
        grid=(b,),
        in_specs=[
            pl.BlockSpec((1, n, f), lambda i: (i, 0, 0)),
            pl.BlockSpec((f, h1), lambda i: (0, 0)),
            pl.BlockSpec((1, h1), lambda i: (0, 0)),
            pl.BlockSpec((h1, h2), lambda i: (0, 0)),
            pl.BlockSpec((1, h2), lambda i: (0, 0)),
        ],
        out_specs=pl.BlockSpec((1, 1, h2), lambda i: (i, 0, 0)),
        out_shape=jax.ShapeDtypeStruct((b, 1, h2), F32),
        compiler_params=_params(1),
    )(table, w1, p['b1'].reshape(1, -1), w2, p['b2'].reshape(1, -1))
    return out[:, 0, :]


def kernel(partial, params):
    b = partial.shape[0]
    partial_t = jnp.transpose(partial, (0, 2, 1))

    new_xyz1 = _fps(partial, 1024)
    idx1 = _knn(new_xyz1, partial)
    table1 = jnp.concatenate([partial_t, partial_t], axis=2)
    nx1_t = jnp.transpose(new_xyz1, (0, 2, 1))
    l1_pre = _sa(table1, nx1_t, idx1, params['sa1'], t=128)

    idx_t1 = _knn(new_xyz1, new_xyz1)
    l1_pts = _vt(l1_pre, nx1_t, idx_t1, params['t1'])

    new_xyz2 = _fps(new_xyz1, 512)
    idx2 = _knn(new_xyz2, new_xyz1)
    table2 = jnp.concatenate([nx1_t, l1_pts], axis=2)
    nx2_t = jnp.transpose(new_xyz2, (0, 2, 1))
    l2_pre = _sa(table2, nx2_t, idx2, params['sa2'], t=128)

    idx_t2 = _knn(new_xyz2, new_xyz2)
    l2_pts = _vt(l2_pre, nx2_t, idx_t2, params['t2'])

    table3 = jnp.concatenate([nx2_t, l2_pts], axis=2)
    l3 = _sa3(table3, params['sa3'])

    l2_points_cf = jnp.transpose(l2_pts, (0, 2, 1))
    n = l2_points_cf.shape[2]
    feat_re = jnp.broadcast_to(l3[:, :, None], (b, l3.shape[1], n))
    out = jnp.concatenate([l2_points_cf, feat_re], axis=1)
    return new_xyz2, out


# speedup vs baseline: 1.0200x; 1.0200x over previous
import functools

import jax
import jax.numpy as jnp
from jax.experimental import pallas as pl
from jax.experimental.pallas import tpu as pltpu
from jax.experimental.pallas import tpu_sc as plsc

F32 = jnp.float32
INF = float('inf')


def _params(grid_len):
    return pltpu.CompilerParams(
        dimension_semantics=("parallel",) * grid_len)


def _fps_body(xyz_ref, out_ref, *, npoint, n, b):
    x = xyz_ref[:, 0, :]
    y = xyz_ref[:, 1, :]
    z = xyz_ref[:, 2, :]
    iota = jax.lax.broadcasted_iota(jnp.int32, (b, n), 1)

    def body(i, carry):
        dists, far = carry
        mask = iota == far
        cx = jnp.sum(jnp.where(mask, x, 0.0), axis=1)
        cy = jnp.sum(jnp.where(mask, y, 0.0), axis=1)
        cz = jnp.sum(jnp.where(mask, z, 0.0), axis=1)
        c = jnp.concatenate([cx[:, None], cy[:, None], cz[:, None]], axis=1)
        out_ref[0, pl.ds(i, 1)] = c[None]
        d = (x - cx[:, None]) ** 2
        d = d + (y - cy[:, None]) ** 2
        d = d + (z - cz[:, None]) ** 2
        dists = jnp.minimum(dists, d)
        far = jnp.argmax(dists, axis=1).astype(jnp.int32)[:, None]
        return dists, far

    dists0 = jnp.full((b, n), 1e10, F32)
    far0 = jnp.zeros((b, 1), jnp.int32)
    jax.lax.fori_loop(0, npoint, body, (dists0, far0))


def _fps(xyz, npoint):
    b, _, n = xyz.shape
    out = pl.pallas_call(
        functools.partial(_fps_body, npoint=npoint, n=n, b=b),
        grid=(1,),
        in_specs=[pl.BlockSpec((b, 3, n), lambda i: (0, 0, 0))],
        out_specs=pl.BlockSpec((1, npoint, b, 3), lambda i: (0, 0, 0, 0)),
        out_shape=jax.ShapeDtypeStruct((1, npoint, b, 3), F32),
    )(xyz)
    return jnp.transpose(out, (0, 2, 3, 1)).reshape(b, 3, npoint)


def _knn_body(q_ref, p_ref, idx_ref, *, k, n, t):
    q = q_ref[0]
    p = p_ref[0]
    q2 = jnp.sum(q * q, axis=1)[:, None]
    p2 = jnp.sum(p * p, axis=1)[None, :]
    d = -2.0 * jnp.dot(q, p.T, preferred_element_type=F32)
    d = d + q2
    d = d + p2
    iota = jax.lax.broadcasted_iota(jnp.int32, (t, n), 1)
    cols = []
    for _ in range(k):
        j = jnp.argmin(d, axis=1).astype(jnp.int32)[:, None]
        cols.append(j)
        d = jnp.where(iota == j, INF, d)
    idx_ref[0] = jnp.concatenate(cols, axis=1) + pl.program_id(0) * n


def _knn(new_xyz, xyz, k=16, t=512):
    b, _, s = new_xyz.shape
    n = xyz.shape[2]
    t = min(t, s)
    qt = jnp.transpose(new_xyz, (0, 2, 1))
    pt = jnp.transpose(xyz, (0, 2, 1))
    return pl.pallas_call(
        functools.partial(_knn_body, k=k, n=n, t=t),
        grid=(b, s // t),
        in_specs=[
            pl.BlockSpec((1, t, 3), lambda i, j: (i, j, 0)),
            pl.BlockSpec((1, n, 3), lambda i, j: (i, 0, 0)),
        ],
        out_specs=pl.BlockSpec((1, t, k), lambda i, j: (i, j, 0)),
        out_shape=jax.ShapeDtypeStruct((b, s, k), jnp.int32),
        compiler_params=_params(2),
    )(qt, pt)


def _sc_gather(table, idx):
    v, d = table.shape
    bsz = idx.shape[0]
    info = plsc.get_sparse_core_info()
    nc = info.num_cores
    nw = nc * info.num_subcores
    b_per_w = bsz // nw
    c = 256 if d <= 128 else 128
    iters = b_per_w // (2 * c)
    mesh = plsc.VectorSubcoreMesh(core_axis_name="c", subcore_axis_name="s")

    @functools.partial(
        pl.kernel, mesh=mesh,
        out_type=jax.ShapeDtypeStruct((bsz, d), F32),
        scratch_types=[
            pltpu.VMEM((c,), jnp.int32),
            pltpu.VMEM((c, d), F32),
            pltpu.VMEM((c,), jnp.int32),
            pltpu.VMEM((c, d), F32),
            pltpu.SemaphoreType.DMA,
            pltpu.SemaphoreType.DMA,
        ],
    )
    def k(table_hbm, idx_hbm, out_hbm, idx_v0, rows_v0, idx_v1, rows_v1,
          sem0, sem1):
        wid = jax.lax.axis_index("s") * nc + jax.lax.axis_index("c")
        base = wid * b_per_w

        def body(it, carry):
            off0 = base + (2 * it) * c
            off1 = off0 + c
            pltpu.sync_copy(idx_hbm.at[pl.ds(off0, c)], idx_v0)
            h0 = pltpu.async_copy(table_hbm.at[idx_v0], rows_v0, sem0)
            pltpu.sync_copy(idx_hbm.at[pl.ds(off1, c)], idx_v1)
            h1 = pltpu.async_copy(table_hbm.at[idx_v1], rows_v1, sem1)
            h0.wait()
            pltpu.sync_copy(rows_v0, out_hbm.at[pl.ds(off0, c)])
            h1.wait()
            pltpu.sync_copy(rows_v1, out_hbm.at[pl.ds(off1, c)])
            return carry

        jax.lax.fori_loop(0, iters, body, 0)

    return k(table, idx)


def _proj_body(table_ref, w1_ref, out_ref, *, pad):
    h = jnp.dot(table_ref[0], w1_ref[:], preferred_element_type=F32)
    if pad:
        h = jnp.concatenate([h, jnp.zeros((h.shape[0], pad), F32)], axis=1)
    out_ref[0] = h


def _proj(table, w1):
    b, n, f = table.shape
    h1 = w1.shape[1]
    h1p = -(-h1 // 128) * 128
    return pl.pallas_call(
        functools.partial(_proj_body, pad=h1p - h1),
        grid=(b,),
        in_specs=[
            pl.BlockSpec((1, n, f), lambda i: (i, 0, 0)),
            pl.BlockSpec((f, h1), lambda i: (0, 0)),
        ],
        out_specs=pl.BlockSpec((1, n, h1p), lambda i: (i, 0, 0)),
        out_shape=jax.ShapeDtypeStruct((b, n, h1p), F32),
        compiler_params=_params(1),
    )(table, w1)


def _sa_post_body(gath_ref, center_ref, w1x_ref, b1_ref, w2_ref, b2_ref,
                  out_ref, *, s, k, t):
    w1x = w1x_ref[:]
    b1 = b1_ref[:]
    w2 = w2_ref[:]
    b2 = b2_ref[:]
    h1 = w1x.shape[1]
    for ti in range(s // t):
        lo = ti * t
        g = gath_ref[0, lo * k:(lo + t) * k, 0:h1]
        center_t = center_ref[0, lo:lo + t, :]
        c1 = jnp.dot(center_t, w1x, preferred_element_type=F32)
        h = jnp.maximum(
            (g.reshape(t, k, h1) - c1[:, None, :]).reshape(t * k, h1)
            + b1, 0.0)
        h2 = jnp.dot(h, w2, preferred_element_type=F32) + b2
        out_ref[0, lo:lo + t, :] = jnp.max(
            h2.reshape(t, k, h2.shape[1]), axis=1)


def _sa(table, center, idx, p, t):
    b, n, f = table.shape
    s = center.shape[1]
    k = idx.shape[2]
    w1 = p['w1'].T
    w2 = p['w2'].T
    h1 = w1.shape[1]
    h2 = w2.shape[1]
    proj = _proj(table, w1)
    h1p = proj.shape[2]
    gath = _sc_gather(proj.reshape(b * n, h1p), idx.reshape(-1))
    gath = gath.reshape(b, s * k, h1p)
    return pl.pallas_call(
        functools.partial(_sa_post_body, s=s, k=k, t=t),
        grid=(b,),
        in_specs=[
            pl.BlockSpec((1, s * k, h1p), lambda i: (i, 0, 0)),
            pl.BlockSpec((1, s, 3), lambda i: (i, 0, 0)),
            pl.BlockSpec((3, h1), lambda i: (0, 0)),
            pl.BlockSpec((1, h1), lambda i: (0, 0)),
            pl.BlockSpec((h1, h2), lambda i: (0, 0)),
            pl.BlockSpec((1, h2), lambda i: (0, 0)),
        ],
        out_specs=pl.BlockSpec((1, s, h2), lambda i: (i, 0, 0)),
        out_shape=jax.ShapeDtypeStruct((b, s, h2), F32),
        compiler_params=_params(1),
    )(gath, center, jnp.asarray(w1)[0:3, :], p['b1'].reshape(1, -1), w2,
      p['b2'].reshape(1, -1))


def _vt_pre_body(x_ref, pos_ref, ls_w_ref, ls_b_ref, k_w_ref, k_b_ref,
                 q_w_ref, q_b_ref, v_w_ref, v_b_ref, pw1_ref,
                 tab_ref, qp_ref):
    x = x_ref[0]
    xl = jnp.dot(x, ls_w_ref[:], preferred_element_type=F32) + ls_b_ref[:]
    key = jnp.dot(xl, k_w_ref[:], preferred_element_type=F32) + k_b_ref[:]
    val = jnp.dot(xl, v_w_ref[:], preferred_element_type=F32) + v_b_ref[:]
    qry = jnp.dot(xl, q_w_ref[:], preferred_element_type=F32) + q_b_ref[:]
    pp = jnp.dot(pos_ref[0], pw1_ref[:], preferred_element_type=F32)
    pad = jnp.zeros((pp.shape[0], 64), F32)
    tab_ref[0] = jnp.concatenate([key, val, pp, pad], axis=1)
    qp_ref[0] = jnp.concatenate([qry, pp], axis=1)


def _vt_post_body(x_ref, qp_ref, gath_ref,
                  pb1_ref, pg_ref, pbb_ref, pw2_ref, pb2_ref,
                  aw1_ref, ab1_ref, ag_ref, abb_ref, aw2_ref, ab2_ref,
                  le_w_ref, le_b_ref, out_ref, agg_s, *, np_, c, k, t):
    for ti in range(np_ // t):
        lo = ti * t
        g = gath_ref[0, lo * k:(lo + t) * k, :]
        kg = g[:, 0:64]
        vg = g[:, 64:128]
        ppg = g[:, 128:192]
        qp_t = qp_ref[0, lo:lo + t, :]
        q_t = qp_t[:, 0:64]
        pp_t = qp_t[:, 64:128]
        qk_rel = (q_t[:, None, :] - kg.reshape(t, k, 64)).reshape(t * k, 64)
        pe = (pp_t[:, None, :] - ppg.reshape(t, k, 64)).reshape(t * k, 64) \
            + pb1_ref[:]
        pe = jnp.maximum(pe * pg_ref[:] + pbb_ref[:], 0.0)
        pe = jnp.dot(pe, pw2_ref[:], preferred_element_type=F32) + pb2_ref[:]
        a = jnp.dot(qk_rel + pe, aw1_ref[:], preferred_element_type=F32) \
            + ab1_ref[:]
        a = jnp.maximum(a * ag_ref[:] + abb_ref[:], 0.0)
        a = jnp.dot(a, aw2_ref[:], preferred_element_type=F32) + ab2_ref[:]
        a3 = a.reshape(t, k, 64)
        mx = jnp.max(a3, axis=1, keepdims=True)
        e = jnp.exp(a3 - mx)
        sm = e / jnp.sum(e, axis=1, keepdims=True)
        vpe = (vg + pe).reshape(t, k, 64)
        agg_s[lo:lo + t, :] = jnp.sum(sm * vpe, axis=1)
    out_ref[0] = jnp.dot(agg_s[:], le_w_ref[:],
                         preferred_element_type=F32) + le_b_ref[:] + x_ref[0]


def _vt(x, pos, idx, p, t=256):
    b, np_, c = x.shape
    k = idx.shape[2]
    pre_ws = [p['ls_w'].T, p['ls_b'].reshape(1, -1),
              p['k_w'].T, p['k_b'].reshape(1, -1),
              p['q_w'].T, p['q_b'].reshape(1, -1),
              p['v_w'].T, p['v_b'].reshape(1, -1),
              p['pos_w1'].T]
    pre_specs = [pl.BlockSpec(w.shape, lambda i: (0, 0)) for w in pre_ws]
    tab, qp = pl.pallas_call(
        _vt_pre_body,
        grid=(b,),
        in_specs=[
            pl.BlockSpec((1, np_, c), lambda i: (i, 0, 0)),
            pl.BlockSpec((1, np_, 3), lambda i: (i, 0, 0)),
        ] + pre_specs,
        out_specs=[
            pl.BlockSpec((1, np_, 256), lambda i: (i, 0, 0)),
            pl.BlockSpec((1, np_, 128), lambda i: (i, 0, 0)),
        ],
        out_shape=[
            jax.ShapeDtypeStruct((b, np_, 256), F32),
            jax.ShapeDtypeStruct((b, np_, 128), F32),
        ],
        compiler_params=_params(1),
    )(x, pos, *pre_ws)

    gath = _sc_gather(tab.reshape(b * np_, 256), idx.reshape(-1))
    gath = gath.reshape(b, np_ * k, 256)

    post_ws = [p['pos_b1'].reshape(1, -1),
               p['pos_bn_g'].reshape(1, -1), p['pos_bn_b'].reshape(1, -1),
               p['pos_w2'].T, p['pos_b2'].reshape(1, -1),
               p['attn_w1'].T, p['attn_b1'].reshape(1, -1),
               p['attn_bn_g'].reshape(1, -1), p['attn_bn_b'].reshape(1, -1),
               p['attn_w2'].T, p['attn_b2'].reshape(1, -1),
               p['le_w'].T, p['le_b'].reshape(1, -1)]
    post_specs = [pl.BlockSpec(w.shape, lambda i: (0, 0)) for w in post_ws]
    return pl.pallas_call(
        functools.partial(_vt_post_body, np_=np_, c=c, k=k, t=t),
        grid=(b,),
        in_specs=[
            pl.BlockSpec((1, np_, c), lambda i: (i, 0, 0)),
            pl.BlockSpec((1, np_, 128), lambda i: (i, 0, 0)),
            pl.BlockSpec((1, np_ * k, 256), lambda i: (i, 0, 0)),
        ] + post_specs,
        out_specs=pl.BlockSpec((1, np_, c), lambda i: (i, 0, 0)),
        out_shape=jax.ShapeDtypeStruct((b, np_, c), F32),
        scratch_shapes=[pltpu.VMEM((np_, 64), F32)],
        compiler_params=_params(1),
    )(x, qp, gath, *post_ws)


def _sa3_body(table_ref, w1_ref, b1_ref, w2_ref, b2_ref, out_ref):
    h = jnp.maximum(
        jnp.dot(table_ref[0], w1_ref[:], preferred_element_type=F32)
        + b1_ref[:], 0.0)
    h2 = jnp.dot(h, w2_ref[:], preferred_element_type=F32) + b2_ref[:]
    out_ref[0] = jnp.max(h2, axis=0, keepdims=True)


def _sa3(table, p):
    b, n, f = table.shape
    w1 = p['w1'].T
    w2 = p['w2'].T
    h1 = w1.shape[1]
    h2 = w2.shape[1]
    out = pl.pallas_call(
        _sa3_body,
        grid=(b,),
        in_specs=[
            pl.BlockSpec((1, n, f), lambda i: (i, 0, 0)),
            pl.BlockSpec((f, h1), lambda i: (0, 0)),
            pl.BlockSpec((1, h1), lambda i: (0, 0)),
            pl.BlockSpec((h1, h2), lambda i: (0, 0)),
            pl.BlockSpec((1, h2), lambda i: (0, 0)),
        ],
        out_specs=pl.BlockSpec((1, 1, h2), lambda i: (i, 0, 0)),
        out_shape=jax.ShapeDtypeStruct((b, 1, h2), F32),
        compiler_params=_params(1),
    )(table, w1, p['b1'].reshape(1, -1), w2, p['b2'].reshape(1, -1))
    return out[:, 0, :]


def kernel(partial, params):
    b = partial.shape[0]
    partial_t = jnp.transpose(partial, (0, 2, 1))

    new_xyz1 = _fps(partial, 1024)
    idx1 = _knn(new_xyz1, partial)
    table1 = jnp.concatenate([partial_t, partial_t], axis=2)
    nx1_t = jnp.transpose(new_xyz1, (0, 2, 1))
    l1_pre = _sa(table1, nx1_t, idx1, params['sa1'], t=128)

    idx_t1 = _knn(new_xyz1, new_xyz1)
    l1_pts = _vt(l1_pre, nx1_t, idx_t1, params['t1'])

    new_xyz2 = _fps(new_xyz1, 512)
    idx2 = _knn(new_xyz2, new_xyz1)
    table2 = jnp.concatenate([nx1_t, l1_pts], axis=2)
    nx2_t = jnp.transpose(new_xyz2, (0, 2, 1))
    l2_pre = _sa(table2, nx2_t, idx2, params['sa2'], t=128)

    idx_t2 = _knn(new_xyz2, new_xyz2)
    l2_pts = _vt(l2_pre, nx2_t, idx_t2, params['t2'])

    table3 = jnp.concatenate([nx2_t, l2_pts], axis=2)
    l3 = _sa3(table3, params['sa3'])

    l2_points_cf = jnp.transpose(l2_pts, (0, 2, 1))
    n = l2_points_cf.shape[2]
    feat_re = jnp.broadcast_to(l3[:, :, None], (b, l3.shape[1], n))
    out = jnp.concatenate([l2_points_cf, feat_re], axis=1)
    return new_xyz2, out
